# Initial kernel scaffold; baseline (speedup 1.0000x reference)
import jax
import jax.numpy as jnp
from jax.experimental import pallas as pl


def kernel(x, edge_index, W, b):
    raise NotImplementedError("write your pallas kernel here")


# speedup vs baseline: 14.6479x; 14.6479x over previous
import functools

import jax
import jax.numpy as jnp
from jax import lax
from jax.experimental import pallas as pl
from jax.experimental.pallas import tpu as pltpu
from jax.experimental.pallas import tpu_sc as plsc

N_NODES = 10000
N_EDGES = 320000
D = 128

NC = 2
NS = 16
NW = NC * NS
LANES = 16

NP = 10240
ROWS_PER_TILE = NP // NS
CB = 128
EPT_CHUNKS = 80
EPT = CB * EPT_CHUNKS
EP = EPT * NW
RB = 1280


def _vmesh():
    return plsc.VectorSubcoreMesh(core_axis_name="c", subcore_axis_name="s")


@functools.partial(
    pl.kernel,
    out_type=jax.ShapeDtypeStruct((NW, NP), jnp.float32),
    mesh=_vmesh(),
    compiler_params=pltpu.CompilerParams(needs_layout_passes=False),
    scratch_types=[
        pltpu.VMEM((EPT_CHUNKS, CB), jnp.int32),
        pltpu.VMEM((NP,), jnp.float32),
    ],
)
def _hist_sc(dst_hbm, out_hbm, didx, hist):
    cid = lax.axis_index("c")
    sid = lax.axis_index("s")
    wid = cid * NS + sid

    pltpu.sync_copy(dst_hbm.at[wid], didx)

    def zrow(i, carry):
        hist[pl.ds(i * LANES, LANES)] = jnp.zeros((LANES,), jnp.float32)
        return carry

    lax.fori_loop(0, NP // LANES, zrow, 0)

    ones = jnp.ones((LANES,), jnp.float32)

    def body(j, carry):
        for k in range(CB // LANES):
            idx = didx[j, pl.ds(k * LANES, LANES)]
            plsc.addupdate_scatter(hist, [idx], ones)
        return carry

    lax.fori_loop(0, EPT_CHUNKS, body, 0)

    pltpu.sync_copy(hist, out_hbm.at[wid])


def _scale_body(x_ref, ht_ref, y_ref, dinv_ref):
    deg = jnp.sum(ht_ref[...], axis=1, keepdims=True) + 1.0
    dinv = lax.rsqrt(deg)
    dinv_ref[...] = dinv
    y_ref[...] = x_ref[...] * dinv


_scale = pl.pallas_call(
    _scale_body,
    grid=(NP // RB,),
    in_specs=[
        pl.BlockSpec((RB, D), lambda i: (i, 0)),
        pl.BlockSpec((RB, NW), lambda i: (i, 0)),
    ],
    out_specs=[
        pl.BlockSpec((RB, D), lambda i: (i, 0)),
        pl.BlockSpec((RB, 1), lambda i: (i, 0)),
    ],
    out_shape=[
        jax.ShapeDtypeStruct((NP, D), jnp.float32),
        jax.ShapeDtypeStruct((NP, 1), jnp.float32),
    ],
)


@functools.partial(
    pl.kernel,
    out_type=jax.ShapeDtypeStruct((NC, NP, D), jnp.float32),
    mesh=_vmesh(),
    scratch_types=[
        pltpu.VMEM((EPT_CHUNKS, CB), jnp.int32),
        pltpu.VMEM((EPT_CHUNKS, CB), jnp.int32),
        pltpu.VMEM((CB, D), jnp.float32),
        pltpu.VMEM((32, D), jnp.float32),
        pltpu.VMEM_SHARED((NP, D), jnp.float32),
        pltpu.SemaphoreType.DMA,
    ],
)
def _scatter_sc(y_hbm, src_hbm, dst_hbm, out_hbm, sidx, didx, rows, zbuf, acc, sem):
    cid = lax.axis_index("c")
    sid = lax.axis_index("s")
    wid = cid * NS + sid
    base = sid * ROWS_PER_TILE

    def zrow(i, carry):
        for k in range(D // LANES):
            zbuf[i, pl.ds(k * LANES, LANES)] = jnp.zeros((LANES,), jnp.float32)
        return carry

    lax.fori_loop(0, 32, zrow, 0)

    def zcp(k, carry):
        pltpu.sync_copy(zbuf, acc.at[pl.ds(base + k * 32, 32)])
        return carry

    lax.fori_loop(0, ROWS_PER_TILE // 32, zcp, 0)

    pltpu.sync_copy(src_hbm.at[wid], sidx)
    pltpu.sync_copy(dst_hbm.at[wid], didx)

    plsc.subcore_barrier()

    def body(j, carry):
        pltpu.async_copy(y_hbm.at[sidx.at[j]], rows, sem).wait()
        pltpu.sync_copy(rows, acc.at[didx.at[j]], add=True)
        return carry

    lax.fori_loop(0, EPT_CHUNKS, body, 0)

    plsc.subcore_barrier()

    pltpu.sync_copy(
        acc.at[pl.ds(base, ROWS_PER_TILE)],
        out_hbm.at[cid, pl.ds(base, ROWS_PER_TILE)],
    )


def _final_body(s0_ref, s1_ref, y_ref, dinv_ref, wt_ref, b_ref, o_ref):
    agg = (s0_ref[...] + s1_ref[...] + y_ref[...]) * dinv_ref[...]
    h = jnp.dot(agg, wt_ref[...], preferred_element_type=jnp.float32)
    o_ref[...] = jnp.maximum(h + b_ref[...], 0.0)


_final = pl.pallas_call(
    _final_body,
    grid=(NP // RB,),
    in_specs=[
        pl.BlockSpec((RB, D), lambda i: (i, 0)),
        pl.BlockSpec((RB, D), lambda i: (i, 0)),
        pl.BlockSpec((RB, D), lambda i: (i, 0)),
        pl.BlockSpec((RB, 1), lambda i: (i, 0)),
        pl.BlockSpec((D, D), lambda i: (0, 0)),
        pl.BlockSpec((1, D), lambda i: (0, 0)),
    ],
    out_specs=pl.BlockSpec((RB, D), lambda i: (i, 0)),
    out_shape=jax.ShapeDtypeStruct((NP, D), jnp.float32),
)


def kernel(x, edge_index, W, b):
    ei = edge_index.astype(jnp.int32)
    pad = jnp.full((EP - N_EDGES,), N_NODES, jnp.int32)
    src3 = jnp.concatenate([ei[0], pad]).reshape(NW, EPT_CHUNKS, CB)
    dst3 = jnp.concatenate([ei[1], pad]).reshape(NW, EPT_CHUNKS, CB)
    x_pad = jnp.pad(x, ((0, NP - N_NODES), (0, 0)))

    hist = _hist_sc(dst3)
    hist_t = hist.T
    y, dinv = _scale(x_pad, hist_t)
    s = _scatter_sc(y, src3, dst3)
    out = _final(s[0], s[1], y, dinv, W.T, jnp.reshape(b, (1, D)))
    return out[:N_NODES]


# speedup vs baseline: 19.7040x; 1.3452x over previous
import functools

import jax
import jax.numpy as jnp
from jax import lax
from jax.experimental import pallas as pl
from jax.experimental.pallas import tpu as pltpu
from jax.experimental.pallas import tpu_sc as plsc

N_NODES = 10000
N_EDGES = 320000
D = 128

NC = 2
NS = 16
NW = NC * NS
LANES = 16

NP = 10240
ROWS_PER_TILE = NP // NS
CB = 128
EPT_CHUNKS = 80
EPT = CB * EPT_CHUNKS
EP = EPT * NW
RB = 1280
NBUF = 4
Q = NP // 4
EPS = EP // NS
EPH = EPS // 2
CROWS = 192
SEC = 32
TRASHI = CROWS * CB - LANES


def _vmesh():
    return plsc.VectorSubcoreMesh(core_axis_name="c", subcore_axis_name="s")


@functools.partial(
    pl.kernel,
    out_type=jax.ShapeDtypeStruct((NW, NP), jnp.float32),
    mesh=_vmesh(),
    compiler_params=pltpu.CompilerParams(needs_layout_passes=False),
    scratch_types=[
        pltpu.VMEM((EPT_CHUNKS, CB), jnp.int32),
        pltpu.VMEM((NP,), jnp.float32),
    ],
)
def _hist_sc(dst_hbm, out_hbm, didx, hist):
    cid = lax.axis_index("c")
    sid = lax.axis_index("s")
    wid = cid * NS + sid

    pltpu.sync_copy(dst_hbm.at[wid], didx)

    def zrow(i, carry):
        hist[pl.ds(i * LANES, LANES)] = jnp.zeros((LANES,), jnp.float32)
        return carry

    lax.fori_loop(0, NP // LANES, zrow, 0)

    ones = jnp.ones((LANES,), jnp.float32)

    def body(j, carry):
        for k in range(CB // LANES):
            idx = didx[j, pl.ds(k * LANES, LANES)]
            plsc.addupdate_scatter(hist, [idx], ones)
        return carry

    lax.fori_loop(0, EPT_CHUNKS, body, 0)

    pltpu.sync_copy(hist, out_hbm.at[wid])


def _scale_body(x_ref, ht_ref, y_ref, dinv_ref):
    deg = jnp.sum(ht_ref[...], axis=1, keepdims=True) + 1.0
    dinv = lax.rsqrt(deg)
    dinv_ref[...] = dinv
    y_ref[...] = x_ref[...] * dinv


_scale = pl.pallas_call(
    _scale_body,
    grid=(NP // RB,),
    in_specs=[
        pl.BlockSpec((RB, D), lambda i: (i, 0)),
        pl.BlockSpec((RB, NW), lambda i: (i, 0)),
    ],
    out_specs=[
        pl.BlockSpec((RB, D), lambda i: (i, 0)),
        pl.BlockSpec((RB, 1), lambda i: (i, 0)),
    ],
    out_shape=[
        jax.ShapeDtypeStruct((NP, D), jnp.float32),
        jax.ShapeDtypeStruct((NP, 1), jnp.float32),
    ],
)


@functools.partial(
    pl.kernel,
    out_type=(
        jax.ShapeDtypeStruct((2, NC, NS, CROWS, CB), jnp.int32),
        jax.ShapeDtypeStruct((2, NC, NS, CROWS, CB), jnp.int32),
        jax.ShapeDtypeStruct((2, NC, NS, LANES), jnp.int32),
    ),
    mesh=_vmesh(),
    compiler_params=pltpu.CompilerParams(needs_layout_passes=False),
    scratch_types=[
        pltpu.VMEM((EPH,), jnp.int32),
        pltpu.VMEM((EPH,), jnp.int32),
        pltpu.VMEM((CROWS, CB), jnp.int32),
        pltpu.VMEM((CROWS, CB), jnp.int32),
        pltpu.VMEM((CROWS, CB), jnp.int32),
        pltpu.VMEM((CROWS, CB), jnp.int32),
        pltpu.VMEM((LANES,), jnp.int32),
    ],
)
def _route_sc(src_hbm, dst_hbm, csrc_hbm, cdst_hbm, cnt_hbm, sraw, draw,
              sidx0, didx0, sidx1, didx1, cbuf):
    cid = lax.axis_index("c")
    sid = lax.axis_index("s")
    lo0 = cid * Q
    lo1 = (2 + cid) * Q

    lanes16 = jnp.zeros((LANES,), jnp.int32)
    trash = jnp.full((LANES,), NP + Q, jnp.int32)
    lane_iota = lax.iota(jnp.int32, LANES)

    def compact(d, s, lo, sidx, didx, n):
        dl = d - lo
        m = (dl >= 0) & (dl < Q)
        mi = m.astype(jnp.int32)
        cum = plsc.cumsum(mi)
        pos = jnp.where(m, n + cum - 1, TRASHI + lane_iota)
        pr = lax.shift_right_logical(pos, 7)
        pc = lax.bitwise_and(pos, CB - 1)
        plsc.store_scatter(sidx, [pr, pc], s)
        plsc.store_scatter(didx, [pr, pc], dl + NP)
        return n + jnp.sum(mi)

    def cbody(v, ns):
        n0, n1 = ns
        s = sraw[pl.ds(v * LANES, LANES)]
        d = draw[pl.ds(v * LANES, LANES)]
        n0 = compact(d, s, lo0, sidx0, didx0, n0)
        n1 = compact(d, s, lo1, sidx1, didx1, n1)
        return (n0, n1)

    ns = (jnp.int32(0), jnp.int32(0))
    for h in range(2):
        pltpu.sync_copy(src_hbm.at[sid, pl.ds(h * EPH, EPH)], sraw)
        pltpu.sync_copy(dst_hbm.at[sid, pl.ds(h * EPH, EPH)], draw)
        ns = lax.fori_loop(0, EPH // LANES, cbody, ns)
    n0, n1 = ns

    for p, n, sidx, didx in ((0, n0, sidx0, didx0), (1, n1, sidx1, didx1)):
        for k in range(CB // LANES):
            pp = n + k * LANES + lane_iota
            pr = lax.shift_right_logical(pp, 7)
            pc = lax.bitwise_and(pp, CB - 1)
            plsc.store_scatter(sidx, [pr, pc], lanes16)
            plsc.store_scatter(didx, [pr, pc], trash)
        pltpu.sync_copy(sidx, csrc_hbm.at[p, cid, sid])
        pltpu.sync_copy(didx, cdst_hbm.at[p, cid, sid])
        cbuf[...] = jnp.full((LANES,), 0, jnp.int32) + n
        pltpu.sync_copy(cbuf, cnt_hbm.at[p, cid, sid])


@functools.partial(
    pl.kernel,
    out_type=jax.ShapeDtypeStruct((NC, Q, D), jnp.float32),
    mesh=_vmesh(),
    scratch_types=[
        pltpu.VMEM((SEC, CB), jnp.int32),
        pltpu.VMEM((SEC, CB), jnp.int32),
        pltpu.VMEM((LANES,), jnp.int32),
        pltpu.VMEM((CB, D), jnp.float32),
        pltpu.VMEM((16, D), jnp.float32),
        pltpu.VMEM_SHARED((NP + Q + 8, D), jnp.float32),
        pltpu.SemaphoreType.DMA,
    ],
)
def _scatter_sc(y_hbm, csrc_hbm, cdst_hbm, cnt_hbm, out_hbm, sidx, didx,
                cbuf, gbuf, zbuf, shared, gsem):
    cid = lax.axis_index("c")
    sid = lax.axis_index("s")

    pltpu.sync_copy(cnt_hbm.at[cid, sid], cbuf)

    ybase = sid * ROWS_PER_TILE
    pltpu.sync_copy(y_hbm.at[pl.ds(ybase, ROWS_PER_TILE)],
                    shared.at[pl.ds(ybase, ROWS_PER_TILE)])

    def zrow(i, carry):
        for k in range(D // LANES):
            zbuf[i, pl.ds(k * LANES, LANES)] = jnp.zeros((LANES,), jnp.float32)
        return carry

    lax.fori_loop(0, 16, zrow, 0)
    zbase = sid * (Q // NS)

    def zcp(k, carry):
        pltpu.sync_copy(zbuf, shared.at[pl.ds(NP + zbase + k * 16, 16)])
        return carry

    lax.fori_loop(0, Q // NS // 16, zcp, 0)

    n = cbuf[...][0]
    nblocks = (n + CB - 1) // CB
    nsec = (nblocks + SEC - 1) // SEC

    plsc.subcore_barrier()

    def sbody(sec, carry):
        pltpu.sync_copy(csrc_hbm.at[cid, sid, pl.ds(sec * SEC, SEC)], sidx)
        pltpu.sync_copy(cdst_hbm.at[cid, sid, pl.ds(sec * SEC, SEC)], didx)
        for j in range(SEC):
            @pl.when(sec * SEC + j < nblocks)
            def _():
                pltpu.async_copy(shared.at[sidx.at[j]], gbuf, gsem).wait()
                pltpu.sync_copy(gbuf, shared.at[didx.at[j]], add=True)
        return carry

    lax.fori_loop(0, nsec, sbody, 0)

    plsc.subcore_barrier()

    obase = sid * (Q // NS)
    pltpu.sync_copy(
        shared.at[pl.ds(NP + obase, Q // NS)],
        out_hbm.at[cid, pl.ds(obase, Q // NS)],
    )


def _final_body(s_ref, y_ref, dinv_ref, wt_ref, b_ref, o_ref):
    agg = (s_ref[...] + y_ref[...]) * dinv_ref[...]
    h = jnp.dot(agg, wt_ref[...], preferred_element_type=jnp.float32)
    o_ref[...] = jnp.maximum(h + b_ref[...], 0.0)


_final = pl.pallas_call(
    _final_body,
    grid=(NP // RB,),
    in_specs=[
        pl.BlockSpec((RB, D), lambda i: (i, 0)),
        pl.BlockSpec((RB, D), lambda i: (i, 0)),
        pl.BlockSpec((RB, 1), lambda i: (i, 0)),
        pl.BlockSpec((D, D), lambda i: (0, 0)),
        pl.BlockSpec((1, D), lambda i: (0, 0)),
    ],
    out_specs=pl.BlockSpec((RB, D), lambda i: (i, 0)),
    out_shape=jax.ShapeDtypeStruct((NP, D), jnp.float32),
)


def kernel(x, edge_index, W, b):
    ei = edge_index.astype(jnp.int32)
    pad = jnp.full((EP - N_EDGES,), N_NODES, jnp.int32)
    src_flat = jnp.concatenate([ei[0], pad])
    dst_flat = jnp.concatenate([ei[1], pad])
    x_pad = jnp.pad(x, ((0, NP - N_NODES), (0, 0)))

    hist = _hist_sc(dst_flat.reshape(NW, EPT_CHUNKS, CB))
    hist_t = hist.T
    y, dinv = _scale(x_pad, hist_t)
    csrc, cdst, cnt = _route_sc(src_flat.reshape(NS, EPS),
                                dst_flat.reshape(NS, EPS))
    s01 = _scatter_sc(y, csrc[0], cdst[0], cnt[0])
    s23 = _scatter_sc(y, csrc[1], cdst[1], cnt[1])
    s = jnp.concatenate([s01.reshape(2 * Q, D), s23.reshape(2 * Q, D)])
    out = _final(s, y, dinv, W.T, jnp.reshape(b, (1, D)))
    return out[:N_NODES]


# speedup vs baseline: 20.8701x; 1.0592x over previous
import functools

import jax
import jax.numpy as jnp
from jax import lax
from jax.experimental import pallas as pl
from jax.experimental.pallas import tpu as pltpu
from jax.experimental.pallas import tpu_sc as plsc

N_NODES = 10000
N_EDGES = 320000
D = 128

NC = 2
NS = 16
NW = NC * NS
LANES = 16

NP = 10240
ROWS_PER_TILE = NP // NS
CB = 128
EPT_CHUNKS = 80
EPT = CB * EPT_CHUNKS
EP = EPT * NW
RB = 1280
NBUF = 4
Q = NP // 4
EPS = EP // NS
EPH = EPS // 2
CROWS = 192
SEC = 32
TRASHI = CROWS * CB - LANES


def _vmesh():
    return plsc.VectorSubcoreMesh(core_axis_name="c", subcore_axis_name="s")


def _scale_body(x_ref, ht_ref, y_ref, dinv_ref):
    deg = jnp.sum(ht_ref[...], axis=1, keepdims=True) + 1.0
    dinv = lax.rsqrt(deg)
    dinv_ref[...] = dinv
    y_ref[...] = x_ref[...] * dinv


_scale = pl.pallas_call(
    _scale_body,
    grid=(NP // RB,),
    in_specs=[
        pl.BlockSpec((RB, D), lambda i: (i, 0)),
        pl.BlockSpec((RB, NS), lambda i: (i, 0)),
    ],
    out_specs=[
        pl.BlockSpec((RB, D), lambda i: (i, 0)),
        pl.BlockSpec((RB, 1), lambda i: (i, 0)),
    ],
    out_shape=[
        jax.ShapeDtypeStruct((NP, D), jnp.float32),
        jax.ShapeDtypeStruct((NP, 1), jnp.float32),
    ],
)


@functools.partial(
    pl.kernel,
    out_type=(
        jax.ShapeDtypeStruct((2, NC, NS, CROWS, CB), jnp.int32),
        jax.ShapeDtypeStruct((2, NC, NS, CROWS, CB), jnp.int32),
        jax.ShapeDtypeStruct((2, NC, NS, LANES), jnp.int32),
        jax.ShapeDtypeStruct((NC, NS, NP), jnp.float32),
    ),
    mesh=_vmesh(),
    compiler_params=pltpu.CompilerParams(needs_layout_passes=False),
    scratch_types=[
        pltpu.VMEM((EPH,), jnp.int32),
        pltpu.VMEM((EPH,), jnp.int32),
        pltpu.VMEM((CROWS, CB), jnp.int32),
        pltpu.VMEM((CROWS, CB), jnp.int32),
        pltpu.VMEM((CROWS, CB), jnp.int32),
        pltpu.VMEM((CROWS, CB), jnp.int32),
        pltpu.VMEM((LANES,), jnp.int32),
        pltpu.VMEM((NP,), jnp.float32),
    ],
)
def _route_sc(src_hbm, dst_hbm, csrc_hbm, cdst_hbm, cnt_hbm, hist_hbm, sraw,
              draw, sidx0, didx0, sidx1, didx1, cbuf, hist):
    cid = lax.axis_index("c")
    sid = lax.axis_index("s")
    lo0 = cid * Q
    lo1 = (2 + cid) * Q

    lanes16 = jnp.zeros((LANES,), jnp.int32)
    trash = jnp.full((LANES,), NP + Q, jnp.int32)
    lane_iota = lax.iota(jnp.int32, LANES)
    ones = jnp.ones((LANES,), jnp.float32)

    def zhist(i, carry):
        hist[pl.ds(i * LANES, LANES)] = jnp.zeros((LANES,), jnp.float32)
        return carry

    lax.fori_loop(0, NP // LANES, zhist, 0)

    def compact(d, s, lo, sidx, didx, n):
        dl = d - lo
        m = (dl >= 0) & (dl < Q)
        mi = m.astype(jnp.int32)
        cum = plsc.cumsum(mi)
        pos = jnp.where(m, n + cum - 1, TRASHI + lane_iota)
        pr = lax.shift_right_logical(pos, 7)
        pc = lax.bitwise_and(pos, CB - 1)
        plsc.store_scatter(sidx, [pr, pc], s)
        plsc.store_scatter(didx, [pr, pc], dl + NP)
        return n + jnp.sum(mi)

    def cbody(v, ns):
        n0, n1 = ns
        s = sraw[pl.ds(v * LANES, LANES)]
        d = draw[pl.ds(v * LANES, LANES)]
        plsc.addupdate_scatter(hist, [d], ones)
        n0 = compact(d, s, lo0, sidx0, didx0, n0)
        n1 = compact(d, s, lo1, sidx1, didx1, n1)
        return (n0, n1)

    ns = (jnp.int32(0), jnp.int32(0))
    for h in range(2):
        pltpu.sync_copy(src_hbm.at[sid, pl.ds(h * EPH, EPH)], sraw)
        pltpu.sync_copy(dst_hbm.at[sid, pl.ds(h * EPH, EPH)], draw)
        ns = lax.fori_loop(0, EPH // LANES, cbody, ns)
    n0, n1 = ns

    for p, n, sidx, didx in ((0, n0, sidx0, didx0), (1, n1, sidx1, didx1)):
        for k in range(CB // LANES):
            pp = n + k * LANES + lane_iota
            pr = lax.shift_right_logical(pp, 7)
            pc = lax.bitwise_and(pp, CB - 1)
            plsc.store_scatter(sidx, [pr, pc], lanes16)
            plsc.store_scatter(didx, [pr, pc], trash)
        pltpu.sync_copy(sidx, csrc_hbm.at[p, cid, sid])
        pltpu.sync_copy(didx, cdst_hbm.at[p, cid, sid])
        cbuf[...] = jnp.full((LANES,), 0, jnp.int32) + n
        pltpu.sync_copy(cbuf, cnt_hbm.at[p, cid, sid])

    pltpu.sync_copy(hist, hist_hbm.at[cid, sid])


@functools.partial(
    pl.kernel,
    out_type=jax.ShapeDtypeStruct((2, NC, Q, D), jnp.float32),
    mesh=_vmesh(),
    scratch_types=[
        pltpu.VMEM((SEC, CB), jnp.int32),
        pltpu.VMEM((SEC, CB), jnp.int32),
        pltpu.VMEM((LANES,), jnp.int32),
        pltpu.VMEM((CB, D), jnp.float32),
        pltpu.VMEM((16, D), jnp.float32),
        pltpu.VMEM_SHARED((NP + Q + 8, D), jnp.float32),
        pltpu.SemaphoreType.DMA,
    ],
)
def _scatter_sc(y_hbm, csrc_hbm, cdst_hbm, cnt_hbm, out_hbm, sidx, didx,
                cbuf, gbuf, zbuf, shared, gsem):
    cid = lax.axis_index("c")
    sid = lax.axis_index("s")

    ybase = sid * ROWS_PER_TILE
    pltpu.sync_copy(y_hbm.at[pl.ds(ybase, ROWS_PER_TILE)],
                    shared.at[pl.ds(ybase, ROWS_PER_TILE)])

    def zrow(i, carry):
        for k in range(D // LANES):
            zbuf[i, pl.ds(k * LANES, LANES)] = jnp.zeros((LANES,), jnp.float32)
        return carry

    lax.fori_loop(0, 16, zrow, 0)
    zbase = sid * (Q // NS)

    for p in range(2):
        pltpu.sync_copy(cnt_hbm.at[p, cid, sid], cbuf)

        def zcp(k, carry):
            pltpu.sync_copy(zbuf, shared.at[pl.ds(NP + zbase + k * 16, 16)])
            return carry

        lax.fori_loop(0, Q // NS // 16, zcp, 0)

        n = cbuf[...][0]
        nblocks = (n + CB - 1) // CB
        nsec = (nblocks + SEC - 1) // SEC

        plsc.subcore_barrier()

        def sbody(sec, carry):
            pltpu.sync_copy(csrc_hbm.at[p, cid, sid, pl.ds(sec * SEC, SEC)],
                            sidx)
            pltpu.sync_copy(cdst_hbm.at[p, cid, sid, pl.ds(sec * SEC, SEC)],
                            didx)
            for j in range(SEC):
                @pl.when(sec * SEC + j < nblocks)
                def _():
                    pltpu.async_copy(shared.at[sidx.at[j]], gbuf, gsem).wait()
                    pltpu.sync_copy(gbuf, shared.at[didx.at[j]], add=True)
            return carry

        lax.fori_loop(0, nsec, sbody, 0)

        plsc.subcore_barrier()

        obase = sid * (Q // NS)
        pltpu.sync_copy(
            shared.at[pl.ds(NP + obase, Q // NS)],
            out_hbm.at[p, cid, pl.ds(obase, Q // NS)],
        )


def _final_body(s_ref, y_ref, dinv_ref, wt_ref, b_ref, o_ref):
    agg = (s_ref[...] + y_ref[...]) * dinv_ref[...]
    h = jnp.dot(agg, wt_ref[...], preferred_element_type=jnp.float32)
    o_ref[...] = jnp.maximum(h + b_ref[...], 0.0)


_final = pl.pallas_call(
    _final_body,
    grid=(NP // RB,),
    in_specs=[
        pl.BlockSpec((RB, D), lambda i: (i, 0)),
        pl.BlockSpec((RB, D), lambda i: (i, 0)),
        pl.BlockSpec((RB, 1), lambda i: (i, 0)),
        pl.BlockSpec((D, D), lambda i: (0, 0)),
        pl.BlockSpec((1, D), lambda i: (0, 0)),
    ],
    out_specs=pl.BlockSpec((RB, D), lambda i: (i, 0)),
    out_shape=jax.ShapeDtypeStruct((NP, D), jnp.float32),
)


def kernel(x, edge_index, W, b):
    ei = edge_index.astype(jnp.int32)
    pad = jnp.full((EP - N_EDGES,), N_NODES, jnp.int32)
    src_flat = jnp.concatenate([ei[0], pad])
    dst_flat = jnp.concatenate([ei[1], pad])
    x_pad = jnp.pad(x, ((0, NP - N_NODES), (0, 0)))

    csrc, cdst, cnt, hist = _route_sc(src_flat.reshape(NS, EPS),
                                      dst_flat.reshape(NS, EPS))
    hist_t = hist[0].reshape(NS, NP).T
    y, dinv = _scale(x_pad, hist_t)
    sq = _scatter_sc(y, csrc, cdst, cnt)
    s = sq.reshape(NP, D)
    out = _final(s, y, dinv, W.T, jnp.reshape(b, (1, D)))
    return out[:N_NODES]


# speedup vs baseline: 22.4337x; 1.0749x over previous
import functools

import jax
import jax.numpy as jnp
from jax import lax
from jax.experimental import pallas as pl
from jax.experimental.pallas import tpu as pltpu
from jax.experimental.pallas import tpu_sc as plsc

N_NODES = 10000
N_EDGES = 320000
D = 128

NC = 2
NS = 16
NW = NC * NS
LANES = 16

NP = 10240
ROWS_PER_TILE = NP // NS
CB = 128
EPT_CHUNKS = 80
EPT = CB * EPT_CHUNKS
EP = EPT * NW
RB = 1280
NBUF = 4
Q = NP // 4
EPS = EP // NS
EPH = EPS // 2
CROWS = 192
SB = 64
SEC = 16
TRASHI = CROWS * CB - LANES


def _vmesh():
    return plsc.VectorSubcoreMesh(core_axis_name="c", subcore_axis_name="s")


def _scale_body(x_ref, ht_ref, y_ref, dinv_ref):
    deg = jnp.sum(ht_ref[...], axis=1, keepdims=True) + 1.0
    dinv = lax.rsqrt(deg)
    dinv_ref[...] = dinv
    y_ref[...] = x_ref[...] * dinv


_scale = pl.pallas_call(
    _scale_body,
    grid=(NP // RB,),
    in_specs=[
        pl.BlockSpec((RB, D), lambda i: (i, 0)),
        pl.BlockSpec((RB, NS), lambda i: (i, 0)),
    ],
    out_specs=[
        pl.BlockSpec((RB, D), lambda i: (i, 0)),
        pl.BlockSpec((RB, 1), lambda i: (i, 0)),
    ],
    out_shape=[
        jax.ShapeDtypeStruct((NP, D), jnp.float32),
        jax.ShapeDtypeStruct((NP, 1), jnp.float32),
    ],
)


@functools.partial(
    pl.kernel,
    out_type=(
        jax.ShapeDtypeStruct((2, NC, NS, CROWS, CB), jnp.int32),
        jax.ShapeDtypeStruct((2, NC, NS, CROWS, CB), jnp.int32),
        jax.ShapeDtypeStruct((2, NC, NS, LANES), jnp.int32),
        jax.ShapeDtypeStruct((NC, NS, NP), jnp.float32),
    ),
    mesh=_vmesh(),
    compiler_params=pltpu.CompilerParams(needs_layout_passes=False),
    scratch_types=[
        pltpu.VMEM((EPH,), jnp.int32),
        pltpu.VMEM((EPH,), jnp.int32),
        pltpu.VMEM((CROWS, CB), jnp.int32),
        pltpu.VMEM((CROWS, CB), jnp.int32),
        pltpu.VMEM((CROWS, CB), jnp.int32),
        pltpu.VMEM((CROWS, CB), jnp.int32),
        pltpu.VMEM((LANES,), jnp.int32),
        pltpu.VMEM((NP,), jnp.float32),
    ],
)
def _route_sc(src_hbm, dst_hbm, csrc_hbm, cdst_hbm, cnt_hbm, hist_hbm, sraw,
              draw, sidx0, didx0, sidx1, didx1, cbuf, hist):
    cid = lax.axis_index("c")
    sid = lax.axis_index("s")
    lo0 = cid * Q
    lo1 = (2 + cid) * Q

    lanes16 = jnp.zeros((LANES,), jnp.int32)
    trash = jnp.full((LANES,), NP + Q, jnp.int32)
    lane_iota = lax.iota(jnp.int32, LANES)
    ones = jnp.ones((LANES,), jnp.float32)

    def zhist(i, carry):
        hist[pl.ds(i * LANES, LANES)] = jnp.zeros((LANES,), jnp.float32)
        return carry

    lax.fori_loop(0, NP // LANES, zhist, 0)

    def compact(d, s, lo, sidx, didx, n):
        dl = d - lo
        m = (dl >= 0) & (dl < Q)
        mi = m.astype(jnp.int32)
        cum = plsc.cumsum(mi)
        pos = jnp.where(m, n + cum - 1, TRASHI + lane_iota)
        pr = lax.shift_right_logical(pos, 7)
        pc = lax.bitwise_and(pos, CB - 1)
        plsc.store_scatter(sidx, [pr, pc], s)
        plsc.store_scatter(didx, [pr, pc], dl + NP)
        return n + jnp.sum(mi)

    def cbody(v, ns):
        n0, n1 = ns
        s = sraw[pl.ds(v * LANES, LANES)]
        d = draw[pl.ds(v * LANES, LANES)]
        plsc.addupdate_scatter(hist, [d], ones)
        n0 = compact(d, s, lo0, sidx0, didx0, n0)
        n1 = compact(d, s, lo1, sidx1, didx1, n1)
        return (n0, n1)

    ns = (jnp.int32(0), jnp.int32(0))
    for h in range(2):
        pltpu.sync_copy(src_hbm.at[sid, pl.ds(h * EPH, EPH)], sraw)
        pltpu.sync_copy(dst_hbm.at[sid, pl.ds(h * EPH, EPH)], draw)
        ns = lax.fori_loop(0, EPH // LANES, cbody, ns)
    n0, n1 = ns

    for p, n, sidx, didx in ((0, n0, sidx0, didx0), (1, n1, sidx1, didx1)):
        for k in range(CB // LANES):
            pp = n + k * LANES + lane_iota
            pr = lax.shift_right_logical(pp, 7)
            pc = lax.bitwise_and(pp, CB - 1)
            plsc.store_scatter(sidx, [pr, pc], lanes16)
            plsc.store_scatter(didx, [pr, pc], trash)
        pltpu.sync_copy(sidx, csrc_hbm.at[p, cid, sid])
        pltpu.sync_copy(didx, cdst_hbm.at[p, cid, sid])
        cbuf[...] = jnp.full((LANES,), 0, jnp.int32) + n
        pltpu.sync_copy(cbuf, cnt_hbm.at[p, cid, sid])

    pltpu.sync_copy(hist, hist_hbm.at[cid, sid])


@functools.partial(
    pl.kernel,
    out_type=jax.ShapeDtypeStruct((2, NC, Q, D), jnp.float32),
    mesh=_vmesh(),
    scratch_types=[
        pltpu.VMEM((SEC, SB), jnp.int32),
        pltpu.VMEM((SEC, SB), jnp.int32),
        pltpu.VMEM((LANES,), jnp.int32),
        pltpu.VMEM((SB, D), jnp.float32),
        pltpu.VMEM((SB, D), jnp.float32),
        pltpu.VMEM((16, D), jnp.float32),
        pltpu.VMEM_SHARED((NP + Q + 8, D), jnp.float32),
        pltpu.SemaphoreType.DMA((2,)),
    ],
)
def _scatter_sc(y_hbm, csrc_hbm, cdst_hbm, cnt_hbm, out_hbm, sidx, didx,
                cbuf, gbuf0, gbuf1, zbuf, shared, gsem):
    cid = lax.axis_index("c")
    sid = lax.axis_index("s")

    ybase = sid * ROWS_PER_TILE
    pltpu.sync_copy(y_hbm.at[pl.ds(ybase, ROWS_PER_TILE)],
                    shared.at[pl.ds(ybase, ROWS_PER_TILE)])

    def zrow(i, carry):
        for k in range(D // LANES):
            zbuf[i, pl.ds(k * LANES, LANES)] = jnp.zeros((LANES,), jnp.float32)
        return carry

    lax.fori_loop(0, 16, zrow, 0)
    zbase = sid * (Q // NS)

    for p in range(2):
        pltpu.sync_copy(cnt_hbm.at[p, cid, sid], cbuf)

        def zcp(k, carry):
            pltpu.sync_copy(zbuf, shared.at[pl.ds(NP + zbase + k * 16, 16)])
            return carry

        lax.fori_loop(0, Q // NS // 16, zcp, 0)

        n = cbuf[...][0]
        nblocks = (n + SB - 1) // SB
        nsec = (nblocks + SEC - 1) // SEC
        bufs = (gbuf0, gbuf1)

        plsc.subcore_barrier()

        def sbody(sec, carry):
            pltpu.sync_copy(csrc_hbm.at[p, cid, sid, pl.ds(sec * SEC, SEC)],
                            sidx)
            pltpu.sync_copy(cdst_hbm.at[p, cid, sid, pl.ds(sec * SEC, SEC)],
                            didx)
            base = sec * SEC

            @pl.when(base < nblocks)
            def _():
                pltpu.async_copy(shared.at[sidx.at[0]], bufs[0], gsem.at[0])

            for j in range(SEC):
                b = j % 2

                @pl.when(base + j < nblocks)
                def _():
                    pltpu.make_async_copy(y_hbm.at[pl.ds(0, SB)], bufs[b],
                                          gsem.at[b]).wait()
                    if j + 1 < SEC:
                        @pl.when(base + j + 1 < nblocks)
                        def _():
                            pltpu.async_copy(shared.at[sidx.at[j + 1]],
                                             bufs[1 - b], gsem.at[1 - b])
                    pltpu.sync_copy(bufs[b], shared.at[didx.at[j]], add=True)
            return carry

        lax.fori_loop(0, nsec, sbody, 0)

        plsc.subcore_barrier()

        obase = sid * (Q // NS)
        pltpu.sync_copy(
            shared.at[pl.ds(NP + obase, Q // NS)],
            out_hbm.at[p, cid, pl.ds(obase, Q // NS)],
        )


def _final_body(s_ref, y_ref, dinv_ref, wt_ref, b_ref, o_ref):
    agg = (s_ref[...] + y_ref[...]) * dinv_ref[...]
    h = jnp.dot(agg, wt_ref[...], preferred_element_type=jnp.float32)
    o_ref[...] = jnp.maximum(h + b_ref[...], 0.0)


_final = pl.pallas_call(
    _final_body,
    grid=(NP // RB,),
    in_specs=[
        pl.BlockSpec((RB, D), lambda i: (i, 0)),
        pl.BlockSpec((RB, D), lambda i: (i, 0)),
        pl.BlockSpec((RB, 1), lambda i: (i, 0)),
        pl.BlockSpec((D, D), lambda i: (0, 0)),
        pl.BlockSpec((1, D), lambda i: (0, 0)),
    ],
    out_specs=pl.BlockSpec((RB, D), lambda i: (i, 0)),
    out_shape=jax.ShapeDtypeStruct((NP, D), jnp.float32),
)


def kernel(x, edge_index, W, b):
    ei = edge_index.astype(jnp.int32)
    pad = jnp.full((EP - N_EDGES,), N_NODES, jnp.int32)
    src_flat = jnp.concatenate([ei[0], pad])
    dst_flat = jnp.concatenate([ei[1], pad])
    x_pad = jnp.pad(x, ((0, NP - N_NODES), (0, 0)))

    csrc, cdst, cnt, hist = _route_sc(src_flat.reshape(NS, EPS),
                                      dst_flat.reshape(NS, EPS))
    hist_t = hist[0].reshape(NS, NP).T
    y, dinv = _scale(x_pad, hist_t)
    csrc2 = csrc.reshape(2, NC, NS, CROWS * CB // SB, SB)
    cdst2 = cdst.reshape(2, NC, NS, CROWS * CB // SB, SB)
    sq = _scatter_sc(y, csrc2, cdst2, cnt)
    s = sq.reshape(NP, D)
    out = _final(s, y, dinv, W.T, jnp.reshape(b, (1, D)))
    return out[:N_NODES]


# speedup vs baseline: 22.4544x; 1.0009x over previous
import functools

import jax
import jax.numpy as jnp
from jax import lax
from jax.experimental import pallas as pl
from jax.experimental.pallas import tpu as pltpu
from jax.experimental.pallas import tpu_sc as plsc

N_NODES = 10000
N_EDGES = 320000
D = 128

NC = 2
NS = 16
NW = NC * NS
LANES = 16

NP = 10240
ROWS_PER_TILE = NP // NS
CB = 128
EPT_CHUNKS = 80
EPT = CB * EPT_CHUNKS
EP = EPT * NW
RB = 1280
NBUF = 4
Q = NP // 4
EPS = EP // NS
EPH = EPS // 2
CROWS = 192
SB = 64
SEC = 16
TRASHI = CROWS * CB - LANES


def _vmesh():
    return plsc.VectorSubcoreMesh(core_axis_name="c", subcore_axis_name="s")


def _scale_body(x_ref, ht_ref, y_ref, dinv_ref):
    deg = jnp.sum(ht_ref[...], axis=1, keepdims=True) + 1.0
    dinv = lax.rsqrt(deg)
    dinv_ref[...] = dinv
    y_ref[...] = x_ref[...] * dinv


_scale = pl.pallas_call(
    _scale_body,
    grid=(NP // RB,),
    in_specs=[
        pl.BlockSpec((RB, D), lambda i: (i, 0)),
        pl.BlockSpec((RB, NS), lambda i: (i, 0)),
    ],
    out_specs=[
        pl.BlockSpec((RB, D), lambda i: (i, 0)),
        pl.BlockSpec((RB, 1), lambda i: (i, 0)),
    ],
    out_shape=[
        jax.ShapeDtypeStruct((NP, D), jnp.float32),
        jax.ShapeDtypeStruct((NP, 1), jnp.float32),
    ],
)


@functools.partial(
    pl.kernel,
    out_type=(
        jax.ShapeDtypeStruct((2, NC, NS, CROWS, CB), jnp.int32),
        jax.ShapeDtypeStruct((2, NC, NS, CROWS, CB), jnp.int32),
        jax.ShapeDtypeStruct((2, NC, NS, LANES), jnp.int32),
        jax.ShapeDtypeStruct((NC, NS, NP), jnp.float32),
    ),
    mesh=_vmesh(),
    compiler_params=pltpu.CompilerParams(needs_layout_passes=False),
    scratch_types=[
        pltpu.VMEM((EPH,), jnp.int32),
        pltpu.VMEM((EPH,), jnp.int32),
        pltpu.VMEM((CROWS, CB), jnp.int32),
        pltpu.VMEM((CROWS, CB), jnp.int32),
        pltpu.VMEM((CROWS, CB), jnp.int32),
        pltpu.VMEM((CROWS, CB), jnp.int32),
        pltpu.VMEM((LANES,), jnp.int32),
        pltpu.VMEM((NP,), jnp.float32),
    ],
)
def _route_sc(src_hbm, dst_hbm, csrc_hbm, cdst_hbm, cnt_hbm, hist_hbm, sraw,
              draw, sidx0, didx0, sidx1, didx1, cbuf, hist):
    cid = lax.axis_index("c")
    sid = lax.axis_index("s")
    lo0 = cid * Q
    lo1 = (2 + cid) * Q

    lanes16 = jnp.zeros((LANES,), jnp.int32)
    trash = jnp.full((LANES,), NP + Q, jnp.int32)
    lane_iota = lax.iota(jnp.int32, LANES)
    ones = jnp.ones((LANES,), jnp.float32)

    def zhist(i, carry):
        hist[pl.ds(i * LANES, LANES)] = jnp.zeros((LANES,), jnp.float32)
        return carry

    lax.fori_loop(0, NP // LANES, zhist, 0)

    def compact(d, s, lo, sidx, didx, n):
        dl = d - lo
        m = (dl >= 0) & (dl < Q)
        mi = m.astype(jnp.int32)
        cum = plsc.cumsum(mi)
        pos = jnp.where(m, n + cum - 1, TRASHI + lane_iota)
        pr = lax.shift_right_logical(pos, 7)
        pc = lax.bitwise_and(pos, CB - 1)
        plsc.store_scatter(sidx, [pr, pc], s)
        plsc.store_scatter(didx, [pr, pc], dl + NP)
        return n + jnp.sum(mi)

    def cbody(v, ns):
        n0, n1 = ns
        s = sraw[pl.ds(v * LANES, LANES)]
        d = draw[pl.ds(v * LANES, LANES)]
        plsc.addupdate_scatter(hist, [d], ones)
        n0 = compact(d, s, lo0, sidx0, didx0, n0)
        n1 = compact(d, s, lo1, sidx1, didx1, n1)
        return (n0, n1)

    ns = (jnp.int32(0), jnp.int32(0))
    for h in range(2):
        pltpu.sync_copy(src_hbm.at[sid, pl.ds(h * EPH, EPH)], sraw)
        pltpu.sync_copy(dst_hbm.at[sid, pl.ds(h * EPH, EPH)], draw)
        ns = lax.fori_loop(0, EPH // LANES, cbody, ns)
    n0, n1 = ns

    for p, n, sidx, didx in ((0, n0, sidx0, didx0), (1, n1, sidx1, didx1)):
        for k in range(CB // LANES):
            pp = n + k * LANES + lane_iota
            pr = lax.shift_right_logical(pp, 7)
            pc = lax.bitwise_and(pp, CB - 1)
            plsc.store_scatter(sidx, [pr, pc], lanes16)
            plsc.store_scatter(didx, [pr, pc], trash)
        pltpu.sync_copy(sidx, csrc_hbm.at[p, cid, sid])
        pltpu.sync_copy(didx, cdst_hbm.at[p, cid, sid])
        cbuf[...] = jnp.full((LANES,), 0, jnp.int32) + n
        pltpu.sync_copy(cbuf, cnt_hbm.at[p, cid, sid])

    pltpu.sync_copy(hist, hist_hbm.at[cid, sid])


@functools.partial(
    pl.kernel,
    out_type=jax.ShapeDtypeStruct((2, NC, Q, D), jnp.float32),
    mesh=_vmesh(),
    scratch_types=[
        pltpu.VMEM((SEC, SB), jnp.int32),
        pltpu.VMEM((SEC, SB), jnp.int32),
        pltpu.VMEM((LANES,), jnp.int32),
        pltpu.VMEM((SB, D), jnp.float32),
        pltpu.VMEM((SB, D), jnp.float32),
        pltpu.VMEM_SHARED((NP + Q + 8, D), jnp.float32),
        pltpu.SemaphoreType.DMA((2,)),
    ],
)
def _scatter_sc(y_hbm, csrc_hbm, cdst_hbm, cnt_hbm, out_hbm, sidx, didx,
                cbuf, gbuf0, gbuf1, shared, gsem):
    cid = lax.axis_index("c")
    sid = lax.axis_index("s")

    ybase = sid * ROWS_PER_TILE
    pltpu.sync_copy(y_hbm.at[pl.ds(ybase, ROWS_PER_TILE)],
                    shared.at[pl.ds(ybase, ROWS_PER_TILE)])

    zbase = sid * (Q // NS)

    for p in range(2):
        pltpu.sync_copy(cnt_hbm.at[p, cid, sid], cbuf)

        qlo = (2 * p + cid) * Q
        pltpu.sync_copy(y_hbm.at[pl.ds(qlo + zbase, Q // NS)],
                        shared.at[pl.ds(NP + zbase, Q // NS)])

        n = cbuf[...][0]
        nblocks = (n + SB - 1) // SB
        nsec = (nblocks + SEC - 1) // SEC
        bufs = (gbuf0, gbuf1)

        plsc.subcore_barrier()

        def sbody(sec, carry):
            pltpu.sync_copy(csrc_hbm.at[p, cid, sid, pl.ds(sec * SEC, SEC)],
                            sidx)
            pltpu.sync_copy(cdst_hbm.at[p, cid, sid, pl.ds(sec * SEC, SEC)],
                            didx)
            base = sec * SEC

            @pl.when(base < nblocks)
            def _():
                pltpu.async_copy(shared.at[sidx.at[0]], bufs[0], gsem.at[0])

            for j in range(SEC):
                b = j % 2

                @pl.when(base + j < nblocks)
                def _():
                    pltpu.make_async_copy(y_hbm.at[pl.ds(0, SB)], bufs[b],
                                          gsem.at[b]).wait()
                    if j + 1 < SEC:
                        @pl.when(base + j + 1 < nblocks)
                        def _():
                            pltpu.async_copy(shared.at[sidx.at[j + 1]],
                                             bufs[1 - b], gsem.at[1 - b])
                    pltpu.sync_copy(bufs[b], shared.at[didx.at[j]], add=True)
            return carry

        lax.fori_loop(0, nsec, sbody, 0)

        plsc.subcore_barrier()

        obase = sid * (Q // NS)
        pltpu.sync_copy(
            shared.at[pl.ds(NP + obase, Q // NS)],
            out_hbm.at[p, cid, pl.ds(obase, Q // NS)],
        )


def _final_body(s_ref, dinv_ref, wt_ref, b_ref, o_ref):
    agg = s_ref[...] * dinv_ref[...]
    h = jnp.dot(agg, wt_ref[...], preferred_element_type=jnp.float32)
    o_ref[...] = jnp.maximum(h + b_ref[...], 0.0)


_final = pl.pallas_call(
    _final_body,
    grid=(NP // RB,),
    in_specs=[
        pl.BlockSpec((RB, D), lambda i: (i, 0)),
        pl.BlockSpec((RB, 1), lambda i: (i, 0)),
        pl.BlockSpec((D, D), lambda i: (0, 0)),
        pl.BlockSpec((1, D), lambda i: (0, 0)),
    ],
    out_specs=pl.BlockSpec((RB, D), lambda i: (i, 0)),
    out_shape=jax.ShapeDtypeStruct((NP, D), jnp.float32),
)


def kernel(x, edge_index, W, b):
    ei = edge_index.astype(jnp.int32)
    pad = jnp.full((EP - N_EDGES,), N_NODES, jnp.int32)
    src_flat = jnp.concatenate([ei[0], pad])
    dst_flat = jnp.concatenate([ei[1], pad])
    x_pad = jnp.pad(x, ((0, NP - N_NODES), (0, 0)))

    csrc, cdst, cnt, hist = _route_sc(src_flat.reshape(NS, EPS),
                                      dst_flat.reshape(NS, EPS))
    hist_t = hist[0].reshape(NS, NP).T
    y, dinv = _scale(x_pad, hist_t)
    csrc2 = csrc.reshape(2, NC, NS, CROWS * CB // SB, SB)
    cdst2 = cdst.reshape(2, NC, NS, CROWS * CB // SB, SB)
    sq = _scatter_sc(y, csrc2, cdst2, cnt)
    s = sq.reshape(NP, D)
    out = _final(s, dinv, W.T, jnp.reshape(b, (1, D)))
    return out[:N_NODES]


# speedup vs baseline: 22.9155x; 1.0205x over previous
import functools

import jax
import jax.numpy as jnp
from jax import lax
from jax.experimental import pallas as pl
from jax.experimental.pallas import tpu as pltpu
from jax.experimental.pallas import tpu_sc as plsc

N_NODES = 10000
N_EDGES = 320000
D = 128

NC = 2
NS = 16
NW = NC * NS
LANES = 16

NP = 10240
ROWS_PER_TILE = NP // NS
CB = 128
EPT_CHUNKS = 80
EPT = CB * EPT_CHUNKS
EP = EPT * NW
RB = 1280
NBUF = 4
Q = NP // 4
EPS = EP // NS
EPH = EPS // 2
CROWS = 192
SB = 96
SEC = 8
TRASHI = CROWS * CB - LANES


def _vmesh():
    return plsc.VectorSubcoreMesh(core_axis_name="c", subcore_axis_name="s")


def _scale_body(x_ref, ht_ref, y_ref, dinv_ref):
    deg = jnp.sum(ht_ref[...], axis=1, keepdims=True) + 1.0
    dinv = lax.rsqrt(deg)
    dinv_ref[...] = dinv
    y_ref[...] = x_ref[...] * dinv


_scale = pl.pallas_call(
    _scale_body,
    grid=(NP // RB,),
    in_specs=[
        pl.BlockSpec((RB, D), lambda i: (i, 0)),
        pl.BlockSpec((RB, NS), lambda i: (i, 0)),
    ],
    out_specs=[
        pl.BlockSpec((RB, D), lambda i: (i, 0)),
        pl.BlockSpec((RB, 1), lambda i: (i, 0)),
    ],
    out_shape=[
        jax.ShapeDtypeStruct((NP, D), jnp.float32),
        jax.ShapeDtypeStruct((NP, 1), jnp.float32),
    ],
)


@functools.partial(
    pl.kernel,
    out_type=(
        jax.ShapeDtypeStruct((2, NC, NS, CROWS, CB), jnp.int32),
        jax.ShapeDtypeStruct((2, NC, NS, CROWS, CB), jnp.int32),
        jax.ShapeDtypeStruct((2, NC, NS, LANES), jnp.int32),
        jax.ShapeDtypeStruct((NC, NS, NP), jnp.float32),
    ),
    mesh=_vmesh(),
    compiler_params=pltpu.CompilerParams(needs_layout_passes=False),
    scratch_types=[
        pltpu.VMEM((EPH,), jnp.int32),
        pltpu.VMEM((EPH,), jnp.int32),
        pltpu.VMEM((CROWS, CB), jnp.int32),
        pltpu.VMEM((CROWS, CB), jnp.int32),
        pltpu.VMEM((CROWS, CB), jnp.int32),
        pltpu.VMEM((CROWS, CB), jnp.int32),
        pltpu.VMEM((LANES,), jnp.int32),
        pltpu.VMEM((NP,), jnp.float32),
    ],
)
def _route_sc(src_hbm, dst_hbm, csrc_hbm, cdst_hbm, cnt_hbm, hist_hbm, sraw,
              draw, sidx0, didx0, sidx1, didx1, cbuf, hist):
    cid = lax.axis_index("c")
    sid = lax.axis_index("s")
    lo0 = cid * Q
    lo1 = (2 + cid) * Q

    lanes16 = jnp.zeros((LANES,), jnp.int32)
    trash = jnp.full((LANES,), NP + Q, jnp.int32)
    lane_iota = lax.iota(jnp.int32, LANES)
    ones = jnp.ones((LANES,), jnp.float32)

    def zhist(i, carry):
        hist[pl.ds(i * LANES, LANES)] = jnp.zeros((LANES,), jnp.float32)
        return carry

    lax.fori_loop(0, NP // LANES, zhist, 0)

    def compact(d, s, lo, sidx, didx, n):
        dl = d - lo
        m = (dl >= 0) & (dl < Q)
        mi = m.astype(jnp.int32)
        cum = plsc.cumsum(mi)
        pos = jnp.where(m, n + cum - 1, TRASHI + lane_iota)
        pr = lax.shift_right_logical(pos, 7)
        pc = lax.bitwise_and(pos, CB - 1)
        plsc.store_scatter(sidx, [pr, pc], s)
        plsc.store_scatter(didx, [pr, pc], dl + NP)
        return n + jnp.sum(mi)

    def cbody(v, ns):
        n0, n1 = ns
        s = sraw[pl.ds(v * LANES, LANES)]
        d = draw[pl.ds(v * LANES, LANES)]
        plsc.addupdate_scatter(hist, [d], ones)
        n0 = compact(d, s, lo0, sidx0, didx0, n0)
        n1 = compact(d, s, lo1, sidx1, didx1, n1)
        return (n0, n1)

    ns = (jnp.int32(0), jnp.int32(0))
    for h in range(2):
        pltpu.sync_copy(src_hbm.at[sid, pl.ds(h * EPH, EPH)], sraw)
        pltpu.sync_copy(dst_hbm.at[sid, pl.ds(h * EPH, EPH)], draw)
        ns = lax.fori_loop(0, EPH // LANES, cbody, ns)
    n0, n1 = ns

    for p, n, sidx, didx in ((0, n0, sidx0, didx0), (1, n1, sidx1, didx1)):
        for k in range(CB // LANES):
            pp = n + k * LANES + lane_iota
            pr = lax.shift_right_logical(pp, 7)
            pc = lax.bitwise_and(pp, CB - 1)
            plsc.store_scatter(sidx, [pr, pc], lanes16)
            plsc.store_scatter(didx, [pr, pc], trash)
        pltpu.sync_copy(sidx, csrc_hbm.at[p, cid, sid])
        pltpu.sync_copy(didx, cdst_hbm.at[p, cid, sid])
        cbuf[...] = jnp.full((LANES,), 0, jnp.int32) + n
        pltpu.sync_copy(cbuf, cnt_hbm.at[p, cid, sid])

    pltpu.sync_copy(hist, hist_hbm.at[cid, sid])


@functools.partial(
    pl.kernel,
    out_type=jax.ShapeDtypeStruct((2, NC, Q, D), jnp.float32),
    mesh=_vmesh(),
    scratch_types=[
        pltpu.VMEM((SEC, SB), jnp.int32),
        pltpu.VMEM((SEC, SB), jnp.int32),
        pltpu.VMEM((LANES,), jnp.int32),
        pltpu.VMEM((SB, D), jnp.float32),
        pltpu.VMEM((SB, D), jnp.float32),
        pltpu.VMEM_SHARED((NP + Q + 8, D), jnp.float32),
        pltpu.SemaphoreType.DMA((2,)),
    ],
)
def _scatter_sc(y_hbm, csrc_hbm, cdst_hbm, cnt_hbm, out_hbm, sidx, didx,
                cbuf, gbuf0, gbuf1, shared, gsem):
    cid = lax.axis_index("c")
    sid = lax.axis_index("s")

    ybase = sid * ROWS_PER_TILE
    pltpu.sync_copy(y_hbm.at[pl.ds(ybase, ROWS_PER_TILE)],
                    shared.at[pl.ds(ybase, ROWS_PER_TILE)])

    zbase = sid * (Q // NS)

    for p in range(2):
        pltpu.sync_copy(cnt_hbm.at[p, cid, sid], cbuf)

        qlo = (2 * p + cid) * Q
        pltpu.sync_copy(y_hbm.at[pl.ds(qlo + zbase, Q // NS)],
                        shared.at[pl.ds(NP + zbase, Q // NS)])

        n = cbuf[...][0]
        nblocks = (n + SB - 1) // SB
        nsec = (nblocks + SEC - 1) // SEC
        bufs = (gbuf0, gbuf1)

        plsc.subcore_barrier()

        def sbody(sec, carry):
            pltpu.sync_copy(csrc_hbm.at[p, cid, sid, pl.ds(sec * SEC, SEC)],
                            sidx)
            pltpu.sync_copy(cdst_hbm.at[p, cid, sid, pl.ds(sec * SEC, SEC)],
                            didx)
            base = sec * SEC

            @pl.when(base < nblocks)
            def _():
                pltpu.async_copy(shared.at[sidx.at[0]], bufs[0], gsem.at[0])

            for j in range(SEC):
                b = j % 2

                @pl.when(base + j < nblocks)
                def _():
                    pltpu.make_async_copy(y_hbm.at[pl.ds(0, SB)], bufs[b],
                                          gsem.at[b]).wait()
                    if j + 1 < SEC:
                        @pl.when(base + j + 1 < nblocks)
                        def _():
                            pltpu.async_copy(shared.at[sidx.at[j + 1]],
                                             bufs[1 - b], gsem.at[1 - b])
                    pltpu.sync_copy(bufs[b], shared.at[didx.at[j]], add=True)
            return carry

        lax.fori_loop(0, nsec, sbody, 0)

        plsc.subcore_barrier()

        obase = sid * (Q // NS)
        pltpu.sync_copy(
            shared.at[pl.ds(NP + obase, Q // NS)],
            out_hbm.at[p, cid, pl.ds(obase, Q // NS)],
        )


def _final_body(s_ref, dinv_ref, wt_ref, b_ref, o_ref):
    agg = s_ref[...] * dinv_ref[...]
    h = jnp.dot(agg, wt_ref[...], preferred_element_type=jnp.float32)
    o_ref[...] = jnp.maximum(h + b_ref[...], 0.0)


_final = pl.pallas_call(
    _final_body,
    grid=(NP // RB,),
    in_specs=[
        pl.BlockSpec((RB, D), lambda i: (i, 0)),
        pl.BlockSpec((RB, 1), lambda i: (i, 0)),
        pl.BlockSpec((D, D), lambda i: (0, 0)),
        pl.BlockSpec((1, D), lambda i: (0, 0)),
    ],
    out_specs=pl.BlockSpec((RB, D), lambda i: (i, 0)),
    out_shape=jax.ShapeDtypeStruct((NP, D), jnp.float32),
)


def kernel(x, edge_index, W, b):
    ei = edge_index.astype(jnp.int32)
    pad = jnp.full((EP - N_EDGES,), N_NODES, jnp.int32)
    src_flat = jnp.concatenate([ei[0], pad])
    dst_flat = jnp.concatenate([ei[1], pad])
    x_pad = jnp.pad(x, ((0, NP - N_NODES), (0, 0)))

    csrc, cdst, cnt, hist = _route_sc(src_flat.reshape(NS, EPS),
                                      dst_flat.reshape(NS, EPS))
    hist_t = hist[0].reshape(NS, NP).T
    y, dinv = _scale(x_pad, hist_t)
    csrc2 = csrc.reshape(2, NC, NS, CROWS * CB // SB, SB)
    cdst2 = cdst.reshape(2, NC, NS, CROWS * CB // SB, SB)
    sq = _scatter_sc(y, csrc2, cdst2, cnt)
    s = sq.reshape(NP, D)
    out = _final(s, dinv, W.T, jnp.reshape(b, (1, D)))
    return out[:N_NODES]
